# Initial kernel scaffold; baseline (speedup 1.0000x reference)
import jax
import jax.numpy as jnp
from jax.experimental import pallas as pl
from jax.experimental.pallas import tpu as pltpu

_CB = 16


def _pool_body(x_ref, o_ref):
    x = x_ref[...]
    h = jnp.maximum(x[:, 0::2, :], x[:, 1::2, :])
    o_ref[...] = jnp.maximum(h[:, :, 0::2], h[:, :, 1::2])


def kernel(x):
    N, C, H, W = x.shape
    NC = N * C
    xf = x.reshape(NC, H, W)
    out = pl.pallas_call(
        _pool_body,
        grid=(NC // _CB,),
        in_specs=[pl.BlockSpec((_CB, H, W), lambda i: (i, 0, 0))],
        out_specs=pl.BlockSpec((_CB, H // 2, W // 2), lambda i: (i, 0, 0)),
        out_shape=jax.ShapeDtypeStruct((NC, H // 2, W // 2), x.dtype),
        compiler_params=pltpu.CompilerParams(
            dimension_semantics=("parallel",),
        ),
    )(xf)
    return out.reshape(N, C, H // 2, W // 2)


# speedup vs baseline: 1.2226x; 1.2226x over previous
import jax
import jax.numpy as jnp
import numpy as np
from jax.experimental import pallas as pl
from jax.experimental.pallas import tpu as pltpu

_R = 256


def _selector() -> np.ndarray:
    s = np.zeros((224, 256), np.float32)
    j = np.arange(112)
    s[2 * j, j] = 1.0
    s[2 * j + 1, j + 128] = 1.0
    return s


def _pool_body(x_ref, s_ref, o_ref, wq_ref):
    v = x_ref[...]
    p = jnp.dot(v, s_ref[...], preferred_element_type=jnp.float32)
    wq_ref[...] = jnp.maximum(p[:, 0:128], p[:, 128:256])
    e = wq_ref[pl.ds(0, _R // 2, 2), :]
    o = wq_ref[pl.ds(1, _R // 2, 2), :]
    o_ref[...] = jnp.maximum(e, o)[:, 0:112]


def kernel(x):
    N, C, H, W = x.shape
    NCH = N * C * H
    Wo = W // 2
    xf = x.reshape(NCH, W)
    s = jnp.asarray(_selector())
    out = pl.pallas_call(
        _pool_body,
        grid=(NCH // _R,),
        in_specs=[
            pl.BlockSpec((_R, W), lambda i: (i, 0)),
            pl.BlockSpec((W, 256), lambda i: (0, 0)),
        ],
        out_specs=pl.BlockSpec((_R // 2, Wo), lambda i: (i, 0)),
        out_shape=jax.ShapeDtypeStruct((NCH // 2, Wo), x.dtype),
        scratch_shapes=[pltpu.VMEM((_R, 128), jnp.float32)],
        compiler_params=pltpu.CompilerParams(
            dimension_semantics=("parallel",),
        ),
    )(xf, s)
    return out.reshape(N, C, H // 2, Wo)


# speedup vs baseline: 3.3439x; 2.7350x over previous
import jax
import jax.numpy as jnp
import numpy as np
from jax.experimental import pallas as pl
from jax.experimental.pallas import tpu as pltpu

_R = 1024


def _selector() -> np.ndarray:
    s = np.zeros((224, 256), np.float32)
    j = np.arange(112)
    s[2 * j, j] = 1.0
    s[2 * j + 1, j + 128] = 1.0
    return s


def _pool_body(x_ref, s_ref, o_ref, wq_ref):
    v = x_ref[...]
    p = jnp.dot(v, s_ref[...], preferred_element_type=jnp.float32)
    wq_ref[...] = jnp.maximum(p[:, 0:128], p[:, 128:256])
    e = wq_ref[pl.ds(0, _R // 2, 2), :]
    o = wq_ref[pl.ds(1, _R // 2, 2), :]
    o_ref[...] = jnp.maximum(e, o)[:, 0:112]


def kernel(x):
    N, C, H, W = x.shape
    NCH = N * C * H
    Wo = W // 2
    xf = x.reshape(NCH, W)
    s = jnp.asarray(_selector())
    out = pl.pallas_call(
        _pool_body,
        grid=(NCH // _R,),
        in_specs=[
            pl.BlockSpec((_R, W), lambda i: (i, 0)),
            pl.BlockSpec((W, 256), lambda i: (0, 0)),
        ],
        out_specs=pl.BlockSpec((_R // 2, Wo), lambda i: (i, 0)),
        out_shape=jax.ShapeDtypeStruct((NCH // 2, Wo), x.dtype),
        scratch_shapes=[pltpu.VMEM((_R, 128), jnp.float32)],
        compiler_params=pltpu.CompilerParams(
            dimension_semantics=("parallel",),
        ),
    )(xf, s)
    return out.reshape(N, C, H // 2, Wo)


# speedup vs baseline: 4.8145x; 1.4398x over previous
import jax
import jax.numpy as jnp
import numpy as np
from jax.experimental import pallas as pl
from jax.experimental.pallas import tpu as pltpu

_R = 2048


def _selector() -> np.ndarray:
    s = np.zeros((224, 256), np.float32)
    j = np.arange(112)
    s[2 * j, j] = 1.0
    s[2 * j + 1, j + 128] = 1.0
    return s


def _pool_body(x_ref, s_ref, o_ref, wq_ref):
    v = x_ref[...]
    p = jnp.dot(v, s_ref[...], preferred_element_type=jnp.float32)
    wq_ref[...] = jnp.maximum(p[:, 0:128], p[:, 128:256])
    e = wq_ref[pl.ds(0, _R // 2, 2), :]
    o = wq_ref[pl.ds(1, _R // 2, 2), :]
    o_ref[...] = jnp.maximum(e, o)[:, 0:112]


def kernel(x):
    N, C, H, W = x.shape
    NCH = N * C * H
    Wo = W // 2
    xf = x.reshape(NCH, W)
    s = jnp.asarray(_selector())
    out = pl.pallas_call(
        _pool_body,
        grid=(NCH // _R,),
        in_specs=[
            pl.BlockSpec((_R, W), lambda i: (i, 0)),
            pl.BlockSpec((W, 256), lambda i: (0, 0)),
        ],
        out_specs=pl.BlockSpec((_R // 2, Wo), lambda i: (i, 0)),
        out_shape=jax.ShapeDtypeStruct((NCH // 2, Wo), x.dtype),
        scratch_shapes=[pltpu.VMEM((_R, 128), jnp.float32)],
        compiler_params=pltpu.CompilerParams(
            dimension_semantics=("parallel",),
        ),
    )(xf, s)
    return out.reshape(N, C, H // 2, Wo)


# speedup vs baseline: 6.5045x; 1.3510x over previous
import jax
import jax.numpy as jnp
import numpy as np
from jax.experimental import pallas as pl
from jax.experimental.pallas import tpu as pltpu

_R = 4096


def _selector() -> np.ndarray:
    s = np.zeros((224, 256), np.float32)
    j = np.arange(112)
    s[2 * j, j] = 1.0
    s[2 * j + 1, j + 128] = 1.0
    return s


def _pool_body(x_ref, s_ref, o_ref, wq_ref):
    v = x_ref[...]
    p = jnp.dot(v, s_ref[...], preferred_element_type=jnp.float32)
    wq_ref[...] = jnp.maximum(p[:, 0:128], p[:, 128:256])
    e = wq_ref[pl.ds(0, _R // 2, 2), :]
    o = wq_ref[pl.ds(1, _R // 2, 2), :]
    o_ref[...] = jnp.maximum(e, o)[:, 0:112]


def kernel(x):
    N, C, H, W = x.shape
    NCH = N * C * H
    Wo = W // 2
    xf = x.reshape(NCH, W)
    s = jnp.asarray(_selector())
    out = pl.pallas_call(
        _pool_body,
        grid=(NCH // _R,),
        in_specs=[
            pl.BlockSpec((_R, W), lambda i: (i, 0)),
            pl.BlockSpec((W, 256), lambda i: (0, 0)),
        ],
        out_specs=pl.BlockSpec((_R // 2, Wo), lambda i: (i, 0)),
        out_shape=jax.ShapeDtypeStruct((NCH // 2, Wo), x.dtype),
        scratch_shapes=[pltpu.VMEM((_R, 128), jnp.float32)],
        compiler_params=pltpu.CompilerParams(
            dimension_semantics=("parallel",),
        ),
    )(xf, s)
    return out.reshape(N, C, H // 2, Wo)


# speedup vs baseline: 7.5076x; 1.1542x over previous
import jax
import jax.numpy as jnp
import numpy as np
from jax.experimental import pallas as pl
from jax.experimental.pallas import tpu as pltpu

_R = 8192


def _selector() -> np.ndarray:
    s = np.zeros((224, 256), np.float32)
    j = np.arange(112)
    s[2 * j, j] = 1.0
    s[2 * j + 1, j + 128] = 1.0
    return s


def _pool_body(x_ref, s_ref, o_ref, wq_ref):
    v = x_ref[...]
    p = jnp.dot(v, s_ref[...], preferred_element_type=jnp.float32)
    wq_ref[...] = jnp.maximum(p[:, 0:128], p[:, 128:256])
    e = wq_ref[pl.ds(0, _R // 2, 2), :]
    o = wq_ref[pl.ds(1, _R // 2, 2), :]
    o_ref[...] = jnp.maximum(e, o)[:, 0:112]


def kernel(x):
    N, C, H, W = x.shape
    NCH = N * C * H
    Wo = W // 2
    xf = x.reshape(NCH, W)
    s = jnp.asarray(_selector())
    out = pl.pallas_call(
        _pool_body,
        grid=(NCH // _R,),
        in_specs=[
            pl.BlockSpec((_R, W), lambda i: (i, 0)),
            pl.BlockSpec((W, 256), lambda i: (0, 0)),
        ],
        out_specs=pl.BlockSpec((_R // 2, Wo), lambda i: (i, 0)),
        out_shape=jax.ShapeDtypeStruct((NCH // 2, Wo), x.dtype),
        scratch_shapes=[pltpu.VMEM((_R, 128), jnp.float32)],
        compiler_params=pltpu.CompilerParams(
            dimension_semantics=("parallel",),
        ),
    )(xf, s)
    return out.reshape(N, C, H // 2, Wo)
